# baseline (speedup 1.0000x reference)
import functools

import jax
import jax.numpy as jnp
from jax import lax
from jax.experimental import pallas as pl
from jax.experimental.pallas import tpu as pltpu

_SLOPE = 0.01
_EPS = 1e-5


def _tap_masks(H, W, L):
    l = lax.broadcasted_iota(jnp.int32, (1, L), 1) % (H * W)
    hh, ww = l // W, l % W
    masks = []
    for dh in range(3):
        for dw in range(3):
            ok = ((hh + dh - 1 >= 0) & (hh + dh - 1 < H)
                  & (ww + dw - 1 >= 0) & (ww + dw - 1 < W))
            masks.append(ok.astype(jnp.float32))
    return masks


def _conv9(packed_ref, w_ref, Co, H, W):
    L = packed_ref.shape[-1]
    masks = _tap_masks(H, W, L)
    t = lax.dot_general(w_ref[...], packed_ref[...], (((0,), (0,)), ((), ())),
                        preferred_element_type=jnp.float32)
    acc = None
    for dh in range(3):
        for dw in range(3):
            tap = dh * 3 + dw
            ts = t[tap * Co:(tap + 1) * Co, :]
            off = W * (dh - 1) + (dw - 1)
            if off:
                ts = pltpu.roll(ts, (-off) % L, axis=1)
            ts = ts * masks[tap]
            acc = ts if acc is None else acc + ts
    return acc


def _store_stats(st_ref, acc, Co):
    s = jnp.sum(acc, axis=1, keepdims=True)
    s2 = jnp.sum(acc * acc, axis=1, keepdims=True)
    st_ref[0, :Co, :] = jnp.broadcast_to(s, (Co, 128))
    st_ref[0, Co:, :] = jnp.broadcast_to(s2, (Co, 128))


def _bn_coeffs(st_ref, g_ref, be_ref, Co, inv_cnt):
    st = jnp.sum(st_ref[...], axis=0)
    mean = st[:Co, 0:1] * inv_cnt
    var = st[Co:, 0:1] * inv_cnt - mean * mean
    scale = g_ref[:, 0:1] * lax.rsqrt(var + _EPS)
    return scale, be_ref[:, 0:1] - mean * scale


def _stage1_kernel(x_ref, w_ref, acc_ref, st_ref, x2_ref, *, nb, Co, H, W):
    HW = H * W
    for i in range(nb):
        x2_ref[:, i * HW:(i + 1) * HW] = x_ref[i]
    acc = _conv9(x2_ref, w_ref, Co, H, W)
    for i in range(nb):
        acc_ref[i] = acc[:, i * HW:(i + 1) * HW]
    _store_stats(st_ref, acc, Co)


def _stage2_kernel(a1_ref, st1_ref, g1_ref, be1_ref, w_ref, acc_ref, st_ref,
                   y2_ref, *, nb, Co, H, W, inv_cnt):
    HW = H * W
    scale, shift = _bn_coeffs(st1_ref, g1_ref, be1_ref, Co, inv_cnt)
    for i in range(nb):
        y = a1_ref[i] * scale + shift
        y2_ref[:, i * HW:(i + 1) * HW] = jnp.where(
            y > 0, y, _SLOPE * y).astype(jnp.bfloat16)
    acc = _conv9(y2_ref, w_ref, Co, H, W)
    for i in range(nb):
        acc_ref[i] = acc[:, i * HW:(i + 1) * HW]
    _store_stats(st_ref, acc, Co)


def _finish_kernel(a2_ref, st2_ref, g2_ref, be2_ref, o_ref, *, Co, inv_cnt):
    scale, shift = _bn_coeffs(st2_ref, g2_ref, be2_ref, Co, inv_cnt)
    y = a2_ref[...] * scale[None] + shift[None]
    o_ref[...] = jnp.where(y > 0, y, _SLOPE * y)


def kernel(x_nchw, w1, b1, g1, be1, w2, b2, g2, be2):
    N, Ci, H, W = x_nchw.shape
    Co = g1.shape[0]
    HW = H * W
    inv_cnt = 1.0 / float(N * HW)

    w1a = w1.reshape(9, Ci, Co).transpose(1, 0, 2).reshape(Ci, 9 * Co)
    w2a = w2.reshape(9, Co, Co).transpose(1, 0, 2).reshape(Co, 9 * Co)
    w1a, w2a = w1a.astype(jnp.bfloat16), w2a.astype(jnp.bfloat16)
    g1f = jnp.tile(g1.reshape(Co, 1), (1, 128))
    be1f = jnp.tile(be1.reshape(Co, 1), (1, 128))
    g2f = jnp.tile(g2.reshape(Co, 1), (1, 128))
    be2f = jnp.tile(be2.reshape(Co, 1), (1, 128))

    par = pltpu.CompilerParams(dimension_semantics=("parallel",))
    xv = x_nchw.reshape(N, Ci, HW).astype(jnp.bfloat16)

    nb = max(N // 8, 1)
    G = N // nb
    acc1, st1 = pl.pallas_call(
        functools.partial(_stage1_kernel, nb=nb, Co=Co, H=H, W=W),
        out_shape=[jax.ShapeDtypeStruct((N, Co, HW), jnp.float32),
                   jax.ShapeDtypeStruct((G, 2 * Co, 128), jnp.float32)],
        grid=(G,),
        in_specs=[pl.BlockSpec((nb, Ci, HW), lambda i: (i, 0, 0)),
                  pl.BlockSpec((Ci, 9 * Co), lambda i: (0, 0))],
        out_specs=[pl.BlockSpec((nb, Co, HW), lambda i: (i, 0, 0)),
                   pl.BlockSpec((1, 2 * Co, 128), lambda i: (i, 0, 0))],
        scratch_shapes=[pltpu.VMEM((Ci, nb * HW), jnp.bfloat16)],
        compiler_params=par,
    )(xv, w1a)

    acc2, st2 = pl.pallas_call(
        functools.partial(_stage2_kernel, nb=nb, Co=Co, H=H, W=W,
                          inv_cnt=inv_cnt),
        out_shape=[jax.ShapeDtypeStruct((N, Co, HW), jnp.float32),
                   jax.ShapeDtypeStruct((G, 2 * Co, 128), jnp.float32)],
        grid=(G,),
        in_specs=[pl.BlockSpec((nb, Co, HW), lambda i: (i, 0, 0)),
                  pl.BlockSpec((G, 2 * Co, 128), lambda i: (0, 0, 0)),
                  pl.BlockSpec((Co, 128), lambda i: (0, 0)),
                  pl.BlockSpec((Co, 128), lambda i: (0, 0)),
                  pl.BlockSpec((Co, 9 * Co), lambda i: (0, 0))],
        out_specs=[pl.BlockSpec((nb, Co, HW), lambda i: (i, 0, 0)),
                   pl.BlockSpec((1, 2 * Co, 128), lambda i: (i, 0, 0))],
        scratch_shapes=[pltpu.VMEM((Co, nb * HW), jnp.bfloat16)],
        compiler_params=par,
    )(acc1, st1, g1f, be1f, w2a)

    out = pl.pallas_call(
        functools.partial(_finish_kernel, Co=Co, inv_cnt=inv_cnt),
        out_shape=jax.ShapeDtypeStruct((N, Co, HW), jnp.float32),
        grid=(G,),
        in_specs=[pl.BlockSpec((nb, Co, HW), lambda i: (i, 0, 0)),
                  pl.BlockSpec((G, 2 * Co, 128), lambda i: (0, 0, 0)),
                  pl.BlockSpec((Co, 128), lambda i: (0, 0)),
                  pl.BlockSpec((Co, 128), lambda i: (0, 0))],
        out_specs=pl.BlockSpec((nb, Co, HW), lambda i: (i, 0, 0)),
        compiler_params=par,
    )(acc2, st2, g2f, be2f)

    return out.reshape(N, Co, H, W)


# speedup vs baseline: 1.1839x; 1.1839x over previous
import functools

import jax
import jax.numpy as jnp
from jax import lax
from jax.experimental import pallas as pl
from jax.experimental.pallas import tpu as pltpu

_SLOPE = 0.01
_EPS = 1e-5


def _tap_masks(H, W, L):
    l = lax.broadcasted_iota(jnp.int32, (1, L), 1) % (H * W)
    hh, ww = l // W, l % W
    masks = []
    for dh in range(3):
        for dw in range(3):
            ok = ((hh + dh - 1 >= 0) & (hh + dh - 1 < H)
                  & (ww + dw - 1 >= 0) & (ww + dw - 1 < W))
            masks.append(ok.astype(jnp.float32))
    return masks


def _conv9(packed_ref, w_ref, Co, H, W):
    L = packed_ref.shape[-1]
    masks = _tap_masks(H, W, L)
    t = lax.dot_general(w_ref[...], packed_ref[...], (((0,), (0,)), ((), ())),
                        preferred_element_type=jnp.float32)
    acc = None
    for dh in range(3):
        for dw in range(3):
            tap = dh * 3 + dw
            ts = t[tap * Co:(tap + 1) * Co, :]
            off = W * (dh - 1) + (dw - 1)
            if off:
                ts = pltpu.roll(ts, (-off) % L, axis=1)
            ts = ts * masks[tap]
            acc = ts if acc is None else acc + ts
    return acc


def _store_stats(st_ref, acc, Co):
    s = jnp.sum(acc, axis=1, keepdims=True)
    s2 = jnp.sum(acc * acc, axis=1, keepdims=True)
    st_ref[0, :Co, :] = jnp.broadcast_to(s, (Co, 128))
    st_ref[0, Co:, :] = jnp.broadcast_to(s2, (Co, 128))


def _bn_coeffs(st_ref, g_ref, be_ref, Co, inv_cnt):
    st = jnp.sum(st_ref[...], axis=0)
    mean = st[:Co, 0:1] * inv_cnt
    var = st[Co:, 0:1] * inv_cnt - mean * mean
    scale = g_ref[:, 0:1] * lax.rsqrt(var + _EPS)
    return scale, be_ref[:, 0:1] - mean * scale


def _stage1_kernel(x_ref, w_ref, acc_ref, st_ref, x2_ref, *, nb, Co, H, W):
    HW = H * W
    for i in range(nb):
        x2_ref[:, i * HW:(i + 1) * HW] = x_ref[i].astype(jnp.bfloat16)
    acc = _conv9(x2_ref, w_ref, Co, H, W)
    for i in range(nb):
        acc_ref[i] = acc[:, i * HW:(i + 1) * HW]
    _store_stats(st_ref, acc, Co)


def _stage2_kernel(a1_ref, st1_ref, g1_ref, be1_ref, w_ref, acc_ref, st_ref,
                   y2_ref, *, nb, Co, H, W, inv_cnt):
    HW = H * W
    scale, shift = _bn_coeffs(st1_ref, g1_ref, be1_ref, Co, inv_cnt)
    for i in range(nb):
        y = a1_ref[i] * scale + shift
        y2_ref[:, i * HW:(i + 1) * HW] = jnp.where(
            y > 0, y, _SLOPE * y).astype(jnp.bfloat16)
    acc = _conv9(y2_ref, w_ref, Co, H, W)
    for i in range(nb):
        acc_ref[i] = acc[:, i * HW:(i + 1) * HW]
    _store_stats(st_ref, acc, Co)


def _finish_kernel(a2_ref, st2_ref, g2_ref, be2_ref, o_ref, *, Co, inv_cnt):
    scale, shift = _bn_coeffs(st2_ref, g2_ref, be2_ref, Co, inv_cnt)
    y = a2_ref[...] * scale[None] + shift[None]
    o_ref[...] = jnp.where(y > 0, y, _SLOPE * y)


def kernel(x_nchw, w1, b1, g1, be1, w2, b2, g2, be2):
    N, Ci, H, W = x_nchw.shape
    Co = g1.shape[0]
    HW = H * W
    inv_cnt = 1.0 / float(N * HW)

    w1a = w1.reshape(9, Ci, Co).transpose(1, 0, 2).reshape(Ci, 9 * Co)
    w2a = w2.reshape(9, Co, Co).transpose(1, 0, 2).reshape(Co, 9 * Co)
    w1a, w2a = w1a.astype(jnp.bfloat16), w2a.astype(jnp.bfloat16)
    g1f = jnp.tile(g1.reshape(Co, 1), (1, 128))
    be1f = jnp.tile(be1.reshape(Co, 1), (1, 128))
    g2f = jnp.tile(g2.reshape(Co, 1), (1, 128))
    be2f = jnp.tile(be2.reshape(Co, 1), (1, 128))

    par = pltpu.CompilerParams(dimension_semantics=("parallel",))
    xv = x_nchw.reshape(N, Ci, HW)

    nb = max(N // 4, 1)
    G = N // nb
    acc1, st1 = pl.pallas_call(
        functools.partial(_stage1_kernel, nb=nb, Co=Co, H=H, W=W),
        out_shape=[jax.ShapeDtypeStruct((N, Co, HW), jnp.float32),
                   jax.ShapeDtypeStruct((G, 2 * Co, 128), jnp.float32)],
        grid=(G,),
        in_specs=[pl.BlockSpec((nb, Ci, HW), lambda i: (i, 0, 0)),
                  pl.BlockSpec((Ci, 9 * Co), lambda i: (0, 0))],
        out_specs=[pl.BlockSpec((nb, Co, HW), lambda i: (i, 0, 0)),
                   pl.BlockSpec((1, 2 * Co, 128), lambda i: (i, 0, 0))],
        scratch_shapes=[pltpu.VMEM((Ci, nb * HW), jnp.bfloat16)],
        compiler_params=par,
    )(xv, w1a)

    acc2, st2 = pl.pallas_call(
        functools.partial(_stage2_kernel, nb=nb, Co=Co, H=H, W=W,
                          inv_cnt=inv_cnt),
        out_shape=[jax.ShapeDtypeStruct((N, Co, HW), jnp.float32),
                   jax.ShapeDtypeStruct((G, 2 * Co, 128), jnp.float32)],
        grid=(G,),
        in_specs=[pl.BlockSpec((nb, Co, HW), lambda i: (i, 0, 0)),
                  pl.BlockSpec((G, 2 * Co, 128), lambda i: (0, 0, 0)),
                  pl.BlockSpec((Co, 128), lambda i: (0, 0)),
                  pl.BlockSpec((Co, 128), lambda i: (0, 0)),
                  pl.BlockSpec((Co, 9 * Co), lambda i: (0, 0))],
        out_specs=[pl.BlockSpec((nb, Co, HW), lambda i: (i, 0, 0)),
                   pl.BlockSpec((1, 2 * Co, 128), lambda i: (i, 0, 0))],
        scratch_shapes=[pltpu.VMEM((Co, nb * HW), jnp.bfloat16)],
        compiler_params=par,
    )(acc1, st1, g1f, be1f, w2a)

    out = pl.pallas_call(
        functools.partial(_finish_kernel, Co=Co, inv_cnt=inv_cnt),
        out_shape=jax.ShapeDtypeStruct((N, Co, HW), jnp.float32),
        grid=(G,),
        in_specs=[pl.BlockSpec((nb, Co, HW), lambda i: (i, 0, 0)),
                  pl.BlockSpec((G, 2 * Co, 128), lambda i: (0, 0, 0)),
                  pl.BlockSpec((Co, 128), lambda i: (0, 0)),
                  pl.BlockSpec((Co, 128), lambda i: (0, 0))],
        out_specs=pl.BlockSpec((nb, Co, HW), lambda i: (i, 0, 0)),
        compiler_params=par,
    )(acc2, st2, g2f, be2f)

    return out.reshape(N, Co, H, W)


# speedup vs baseline: 1.2057x; 1.0185x over previous
import functools

import jax
import jax.numpy as jnp
from jax import lax
from jax.experimental import pallas as pl
from jax.experimental.pallas import tpu as pltpu

_SLOPE = 0.01
_EPS = 1e-5


def _tap_masks(H, W, L):
    l = lax.broadcasted_iota(jnp.int32, (1, L), 1) % (H * W)
    hh, ww = l // W, l % W
    masks = []
    for dh in range(3):
        for dw in range(3):
            ok = ((hh + dh - 1 >= 0) & (hh + dh - 1 < H)
                  & (ww + dw - 1 >= 0) & (ww + dw - 1 < W))
            masks.append(ok.astype(jnp.float32))
    return masks


def _conv9(packed_ref, w_ref, Co, H, W):
    L = packed_ref.shape[-1]
    masks = _tap_masks(H, W, L)
    t = lax.dot_general(w_ref[...], packed_ref[...], (((0,), (0,)), ((), ())),
                        preferred_element_type=jnp.float32)
    acc = None
    for dh in range(3):
        for dw in range(3):
            tap = dh * 3 + dw
            ts = t[tap * Co:(tap + 1) * Co, :]
            off = W * (dh - 1) + (dw - 1)
            if off:
                ts = pltpu.roll(ts, (-off) % L, axis=1)
            ts = ts * masks[tap]
            acc = ts if acc is None else acc + ts
    return acc


def _store_stats(st_ref, acc, Co):
    s = jnp.sum(acc, axis=1, keepdims=True)
    s2 = jnp.sum(acc * acc, axis=1, keepdims=True)
    st_ref[0, :Co, :] = jnp.broadcast_to(s, (Co, 128))
    st_ref[0, Co:, :] = jnp.broadcast_to(s2, (Co, 128))


def _bn_coeffs(st_ref, g_ref, be_ref, Co, inv_cnt):
    st = jnp.sum(st_ref[...], axis=0)
    mean = st[:Co, 0:1] * inv_cnt
    var = st[Co:, 0:1] * inv_cnt - mean * mean
    scale = g_ref[:, 0:1] * lax.rsqrt(var + _EPS)
    return scale, be_ref[:, 0:1] - mean * scale


def _stage1_kernel(x_ref, w_ref, acc_ref, st_ref, x2_ref, *, nb, Co, H, W):
    HW = H * W
    for i in range(nb):
        x2_ref[:, i * HW:(i + 1) * HW] = x_ref[i].astype(jnp.bfloat16)
    acc = _conv9(x2_ref, w_ref, Co, H, W)
    for i in range(nb):
        acc_ref[i] = acc[:, i * HW:(i + 1) * HW]
    _store_stats(st_ref, acc, Co)


def _stage2_kernel(a1_ref, st1_ref, g1_ref, be1_ref, w_ref, acc_ref, st_ref,
                   y2_ref, *, nb, Co, H, W, inv_cnt):
    HW = H * W
    scale, shift = _bn_coeffs(st1_ref, g1_ref, be1_ref, Co, inv_cnt)
    for i in range(nb):
        y = a1_ref[i] * scale + shift
        y2_ref[:, i * HW:(i + 1) * HW] = jnp.where(
            y > 0, y, _SLOPE * y).astype(jnp.bfloat16)
    acc = _conv9(y2_ref, w_ref, Co, H, W)
    for i in range(nb):
        acc_ref[i] = acc[:, i * HW:(i + 1) * HW]
    _store_stats(st_ref, acc, Co)


def _finish_kernel(a2_ref, st2_ref, g2_ref, be2_ref, o_ref, *, Co, inv_cnt):
    scale, shift = _bn_coeffs(st2_ref, g2_ref, be2_ref, Co, inv_cnt)
    y = a2_ref[...] * scale[None] + shift[None]
    o_ref[...] = jnp.where(y > 0, y, _SLOPE * y)


def kernel(x_nchw, w1, b1, g1, be1, w2, b2, g2, be2):
    N, Ci, H, W = x_nchw.shape
    Co = g1.shape[0]
    HW = H * W
    inv_cnt = 1.0 / float(N * HW)

    w1a = w1.reshape(9, Ci, Co).transpose(1, 0, 2).reshape(Ci, 9 * Co)
    w2a = w2.reshape(9, Co, Co).transpose(1, 0, 2).reshape(Co, 9 * Co)
    w1a, w2a = w1a.astype(jnp.bfloat16), w2a.astype(jnp.bfloat16)
    g1f = jnp.tile(g1.reshape(Co, 1), (1, 128))
    be1f = jnp.tile(be1.reshape(Co, 1), (1, 128))
    g2f = jnp.tile(g2.reshape(Co, 1), (1, 128))
    be2f = jnp.tile(be2.reshape(Co, 1), (1, 128))

    par = pltpu.CompilerParams(dimension_semantics=("parallel",))
    xv = x_nchw.reshape(N, Ci, HW)

    nb = max(N // 2, 1)
    G = N // nb
    acc1, st1 = pl.pallas_call(
        functools.partial(_stage1_kernel, nb=nb, Co=Co, H=H, W=W),
        out_shape=[jax.ShapeDtypeStruct((N, Co, HW), jnp.float32),
                   jax.ShapeDtypeStruct((G, 2 * Co, 128), jnp.float32)],
        grid=(G,),
        in_specs=[pl.BlockSpec((nb, Ci, HW), lambda i: (i, 0, 0)),
                  pl.BlockSpec((Ci, 9 * Co), lambda i: (0, 0))],
        out_specs=[pl.BlockSpec((nb, Co, HW), lambda i: (i, 0, 0)),
                   pl.BlockSpec((1, 2 * Co, 128), lambda i: (i, 0, 0))],
        scratch_shapes=[pltpu.VMEM((Ci, nb * HW), jnp.bfloat16)],
        compiler_params=par,
    )(xv, w1a)

    acc2, st2 = pl.pallas_call(
        functools.partial(_stage2_kernel, nb=nb, Co=Co, H=H, W=W,
                          inv_cnt=inv_cnt),
        out_shape=[jax.ShapeDtypeStruct((N, Co, HW), jnp.float32),
                   jax.ShapeDtypeStruct((G, 2 * Co, 128), jnp.float32)],
        grid=(G,),
        in_specs=[pl.BlockSpec((nb, Co, HW), lambda i: (i, 0, 0)),
                  pl.BlockSpec((G, 2 * Co, 128), lambda i: (0, 0, 0)),
                  pl.BlockSpec((Co, 128), lambda i: (0, 0)),
                  pl.BlockSpec((Co, 128), lambda i: (0, 0)),
                  pl.BlockSpec((Co, 9 * Co), lambda i: (0, 0))],
        out_specs=[pl.BlockSpec((nb, Co, HW), lambda i: (i, 0, 0)),
                   pl.BlockSpec((1, 2 * Co, 128), lambda i: (i, 0, 0))],
        scratch_shapes=[pltpu.VMEM((Co, nb * HW), jnp.bfloat16)],
        compiler_params=par,
    )(acc1, st1, g1f, be1f, w2a)

    out = pl.pallas_call(
        functools.partial(_finish_kernel, Co=Co, inv_cnt=inv_cnt),
        out_shape=jax.ShapeDtypeStruct((N, Co, HW), jnp.float32),
        grid=(G,),
        in_specs=[pl.BlockSpec((nb, Co, HW), lambda i: (i, 0, 0)),
                  pl.BlockSpec((G, 2 * Co, 128), lambda i: (0, 0, 0)),
                  pl.BlockSpec((Co, 128), lambda i: (0, 0)),
                  pl.BlockSpec((Co, 128), lambda i: (0, 0))],
        out_specs=pl.BlockSpec((nb, Co, HW), lambda i: (i, 0, 0)),
        compiler_params=par,
    )(acc2, st2, g2f, be2f)

    return out.reshape(N, Co, H, W)
